# baseline (speedup 1.0000x reference)
import jax
import jax.numpy as jnp
from jax.experimental import pallas as pl

NUM_RANKS = 1000
NUM_CTX = 16
NUM_RANK_TOK = 4
MAX_TOK = 77
DIM = 768
RB = 8


def _body(ctx_ref, rank_ref, sent_ref, out_ref):
    out_ref[...] = sent_ref[...]
    out_ref[:, 1:1 + NUM_CTX, :] = jnp.broadcast_to(
        ctx_ref[...][None], (RB, NUM_CTX, DIM))
    out_ref[:, 1 + NUM_CTX:1 + NUM_CTX + NUM_RANK_TOK, :] = rank_ref[...]


def kernel(context_embeds, rank_embeds, sentence_embeds):
    grid = (NUM_RANKS // RB,)
    return pl.pallas_call(
        _body,
        grid=grid,
        in_specs=[
            pl.BlockSpec((NUM_CTX, DIM), lambda i: (0, 0)),
            pl.BlockSpec((RB, NUM_RANK_TOK, DIM), lambda i: (i, 0, 0)),
            pl.BlockSpec((RB, MAX_TOK, DIM), lambda i: (i, 0, 0)),
        ],
        out_specs=pl.BlockSpec((RB, MAX_TOK, DIM), lambda i: (i, 0, 0)),
        out_shape=jax.ShapeDtypeStruct((NUM_RANKS, MAX_TOK, DIM), jnp.float32),
    )(context_embeds, rank_embeds, sentence_embeds)
